# Initial kernel scaffold; baseline (speedup 1.0000x reference)
import functools

import jax
import jax.numpy as jnp
from jax import lax
from jax.experimental import pallas as pl
from jax.experimental.pallas import tpu as pltpu
from jax.experimental.pallas import tpu_sc as plsc

N = 10000
E = 320000
D_EDGE = 16
D_FEAT = 128
D_GLOB = 128
D_OUT = 128

NUM_CORES = 2
NUM_SUBCORES = 16
NUM_WORKERS = NUM_CORES * NUM_SUBCORES

LANE = 128
ROWS = E // LANE
ROWS_PER_W = ROWS // NUM_WORKERS
ROWS_TAIL = ROWS - ROWS_PER_W * NUM_WORKERS
GROUPS = 6
ROWS_PER_G = ROWS_PER_W // GROUPS
EDGES_PER_G = ROWS_PER_G * LANE
ROWS_PER_TILE = N // NUM_SUBCORES


def _sc_scatter_body(dst_hbm, ea_hbm, agg_out, cnt_out,
                     idx_v, ea_v, ones_v, agg_sh, cnt_sh):
    c = lax.axis_index("c")
    s = lax.axis_index("s")
    w = c * NUM_SUBCORES + s

    zrow = jnp.zeros((D_EDGE,), jnp.float32)
    orow = jnp.ones((D_EDGE,), jnp.float32)

    def _fill(i, _):
        ea_v[i, :] = zrow
        return 0
    lax.fori_loop(0, ROWS_PER_TILE, _fill, 0)

    def _fill1(i, _):
        ones_v[i, :] = orow
        return 0
    lax.fori_loop(0, LANE, _fill1, 0)

    r0 = s * ROWS_PER_TILE
    pltpu.sync_copy(ea_v.at[pl.ds(0, ROWS_PER_TILE), :],
                    agg_sh.at[pl.ds(r0, ROWS_PER_TILE), :])
    pltpu.sync_copy(ea_v.at[pl.ds(0, ROWS_PER_TILE), :],
                    cnt_sh.at[pl.ds(r0, ROWS_PER_TILE), :])
    plsc.subcore_barrier()

    def _group(g, _):
        row0 = w * ROWS_PER_W + g * ROWS_PER_G
        pltpu.sync_copy(dst_hbm.at[pl.ds(row0, ROWS_PER_G), :], idx_v)
        pltpu.sync_copy(ea_hbm.at[pl.ds(row0 * LANE, EDGES_PER_G), :], ea_v)
        for j in range(ROWS_PER_G):
            idx = idx_v.at[j]
            pltpu.sync_copy(ea_v.at[pl.ds(j * LANE, LANE), :],
                            agg_sh.at[idx], add=True)
            pltpu.sync_copy(ones_v, cnt_sh.at[idx], add=True)
        return 0
    lax.fori_loop(0, GROUPS, _group, 0)

    @pl.when(w < ROWS_TAIL)
    def _tail():
        r = NUM_WORKERS * ROWS_PER_W + w
        pltpu.sync_copy(dst_hbm.at[r], idx_v.at[0])
        pltpu.sync_copy(ea_hbm.at[pl.ds(r * LANE, LANE), :],
                        ea_v.at[pl.ds(0, LANE), :])
        pltpu.sync_copy(ea_v.at[pl.ds(0, LANE), :],
                        agg_sh.at[idx_v.at[0]], add=True)
        pltpu.sync_copy(ones_v, cnt_sh.at[idx_v.at[0]], add=True)

    plsc.subcore_barrier()

    pltpu.sync_copy(agg_sh.at[pl.ds(r0, ROWS_PER_TILE), :],
                    agg_out.at[c, pl.ds(r0, ROWS_PER_TILE), :])
    pltpu.sync_copy(cnt_sh.at[pl.ds(r0, ROWS_PER_TILE), :],
                    cnt_out.at[c, pl.ds(r0, ROWS_PER_TILE), :])


@jax.jit
def _sc_scatter(dst2, edge_attr):
    mesh = plsc.VectorSubcoreMesh(core_axis_name="c", subcore_axis_name="s")
    f = pl.kernel(
        _sc_scatter_body,
        mesh=mesh,
        out_type=[
            jax.ShapeDtypeStruct((NUM_CORES, N, D_EDGE), jnp.float32),
            jax.ShapeDtypeStruct((NUM_CORES, N, D_EDGE), jnp.float32),
        ],
        scratch_types=[
            pltpu.VMEM((ROWS_PER_G, LANE), jnp.int32),
            pltpu.VMEM((EDGES_PER_G, D_EDGE), jnp.float32),
            pltpu.VMEM((LANE, D_EDGE), jnp.float32),
            pltpu.VMEM_SHARED((N, D_EDGE), jnp.float32),
            pltpu.VMEM_SHARED((N, D_EDGE), jnp.float32),
        ],
    )
    return f(dst2, edge_attr)


BN = 1000


def _tc_body(pagg_ref, pcnt_ref, x_ref, g_ref, we_ref, wx_ref, wg_ref,
             b_ref, o_ref):
    s = pagg_ref[0] + pagg_ref[1]
    cnt = (pcnt_ref[0, :, :1] + pcnt_ref[1, :, :1])
    mean = s / jnp.maximum(cnt, 1.0)
    gw = jnp.dot(g_ref[...], wg_ref[...], preferred_element_type=jnp.float32)
    out = (jnp.dot(mean, we_ref[...], preferred_element_type=jnp.float32)
           + jnp.dot(x_ref[...], wx_ref[...], preferred_element_type=jnp.float32)
           + gw + b_ref[...])
    o_ref[...] = out


@jax.jit
def _tc_combine(pagg, pcnt, x, g2, w_e, w_x, w_g, b2):
    grid = (N // BN,)
    return pl.pallas_call(
        _tc_body,
        grid=grid,
        in_specs=[
            pl.BlockSpec((NUM_CORES, BN, D_EDGE), lambda i: (0, i, 0)),
            pl.BlockSpec((NUM_CORES, BN, D_EDGE), lambda i: (0, i, 0)),
            pl.BlockSpec((BN, D_FEAT), lambda i: (i, 0)),
            pl.BlockSpec((1, D_GLOB), lambda i: (0, 0)),
            pl.BlockSpec((D_EDGE, D_OUT), lambda i: (0, 0)),
            pl.BlockSpec((D_FEAT, D_OUT), lambda i: (0, 0)),
            pl.BlockSpec((D_GLOB, D_OUT), lambda i: (0, 0)),
            pl.BlockSpec((1, D_OUT), lambda i: (0, 0)),
        ],
        out_specs=pl.BlockSpec((BN, D_OUT), lambda i: (i, 0)),
        out_shape=jax.ShapeDtypeStruct((N, D_OUT), jnp.float32),
    )(pagg, pcnt, x, g2, w_e, w_x, w_g, b2)


def kernel(x, edge_index, edge_attr, global_attr, W, b):
    dst2 = edge_index[1].reshape(ROWS, LANE)
    pagg, pcnt = _sc_scatter(dst2, edge_attr)
    w_e = W[:D_EDGE]
    w_x = W[D_EDGE:D_EDGE + D_FEAT]
    w_g = W[D_EDGE + D_FEAT:]
    g2 = global_attr.reshape(1, D_GLOB)
    b2 = b.reshape(1, D_OUT)
    return _tc_combine(pagg, pcnt, x, g2, w_e, w_x, w_g, b2)


# speedup vs baseline: 6.0190x; 6.0190x over previous
import functools

import jax
import jax.numpy as jnp
from jax import lax
from jax.experimental import pallas as pl
from jax.experimental.pallas import tpu as pltpu
from jax.experimental.pallas import tpu_sc as plsc

N = 10000
E = 320000
D_EDGE = 16
D_FEAT = 128
D_GLOB = 128
D_OUT = 128

NUM_CORES = 2
NUM_SUBCORES = 16
NUM_WORKERS = NUM_CORES * NUM_SUBCORES

LANE = 128
ROWS = E // LANE
ROWS_PER_W = ROWS // NUM_WORKERS
ROWS_TAIL = ROWS - ROWS_PER_W * NUM_WORKERS
GROUPS = 6
ROWS_PER_G = ROWS_PER_W // GROUPS
EDGES_PER_G = ROWS_PER_G * LANE
EXP = 624
EXP_TAIL = N - EXP * NUM_SUBCORES


def _sc_scatter_body(dst_hbm, ea_hbm, agg_out, cnt_out,
                     idx_v, ea_v, ones_v, agg_sh, cnt_sh):
    c = lax.axis_index("c")
    s = lax.axis_index("s")
    w = c * NUM_SUBCORES + s

    zrow = jnp.zeros((D_EDGE,), jnp.float32)
    orow = jnp.ones((D_EDGE,), jnp.float32)

    def _fill(i, _):
        ea_v[i, :] = zrow
        return 0
    lax.fori_loop(0, EXP, _fill, 0)

    def _fill1(i, _):
        ones_v[i, :] = orow
        return 0
    lax.fori_loop(0, LANE, _fill1, 0)

    r0 = s * EXP
    pltpu.sync_copy(ea_v.at[pl.ds(0, EXP), :], agg_sh.at[pl.ds(r0, EXP), :])
    pltpu.sync_copy(ea_v.at[pl.ds(0, EXP), :], cnt_sh.at[pl.ds(r0, EXP), :])

    @pl.when(s == 0)
    def _zero_tail():
        t0 = EXP * NUM_SUBCORES
        pltpu.sync_copy(ea_v.at[pl.ds(0, EXP_TAIL), :],
                        agg_sh.at[pl.ds(t0, EXP_TAIL), :])
        pltpu.sync_copy(ea_v.at[pl.ds(0, EXP_TAIL), :],
                        cnt_sh.at[pl.ds(t0, EXP_TAIL), :])

    plsc.subcore_barrier()

    def _group(g, _):
        row0 = w * ROWS_PER_W + g * ROWS_PER_G
        for j in range(ROWS_PER_G):
            pltpu.sync_copy(dst_hbm.at[pl.ds((row0 + j) * LANE, LANE)],
                            idx_v.at[j])
        pltpu.sync_copy(ea_hbm.at[pl.ds(row0 * LANE, EDGES_PER_G), :], ea_v)
        for j in range(ROWS_PER_G):
            idx = idx_v.at[j]
            pltpu.sync_copy(ea_v.at[pl.ds(j * LANE, LANE), :],
                            agg_sh.at[idx], add=True)
            pltpu.sync_copy(ones_v, cnt_sh.at[idx], add=True)
        return 0
    lax.fori_loop(0, GROUPS, _group, 0)

    @pl.when(w < ROWS_TAIL)
    def _tail():
        r = NUM_WORKERS * ROWS_PER_W + w
        pltpu.sync_copy(dst_hbm.at[pl.ds(r * LANE, LANE)], idx_v.at[0])
        pltpu.sync_copy(ea_hbm.at[pl.ds(r * LANE, LANE), :],
                        ea_v.at[pl.ds(0, LANE), :])
        pltpu.sync_copy(ea_v.at[pl.ds(0, LANE), :],
                        agg_sh.at[idx_v.at[0]], add=True)
        pltpu.sync_copy(ones_v, cnt_sh.at[idx_v.at[0]], add=True)

    plsc.subcore_barrier()

    pltpu.sync_copy(agg_sh.at[pl.ds(r0, EXP), :],
                    agg_out.at[c, pl.ds(r0, EXP), :])
    pltpu.sync_copy(cnt_sh.at[pl.ds(r0, EXP), :],
                    cnt_out.at[c, pl.ds(r0, EXP), :])

    @pl.when(s == 0)
    def _export_tail():
        t0 = EXP * NUM_SUBCORES
        pltpu.sync_copy(agg_sh.at[pl.ds(t0, EXP_TAIL), :],
                        agg_out.at[c, pl.ds(t0, EXP_TAIL), :])
        pltpu.sync_copy(cnt_sh.at[pl.ds(t0, EXP_TAIL), :],
                        cnt_out.at[c, pl.ds(t0, EXP_TAIL), :])


@jax.jit
def _sc_scatter(dst1, edge_attr):
    mesh = plsc.VectorSubcoreMesh(core_axis_name="c", subcore_axis_name="s")
    f = pl.kernel(
        _sc_scatter_body,
        mesh=mesh,
        out_type=[
            jax.ShapeDtypeStruct((NUM_CORES, N, D_EDGE), jnp.float32),
            jax.ShapeDtypeStruct((NUM_CORES, N, D_EDGE), jnp.float32),
        ],
        scratch_types=[
            pltpu.VMEM((ROWS_PER_G, LANE), jnp.int32),
            pltpu.VMEM((EDGES_PER_G, D_EDGE), jnp.float32),
            pltpu.VMEM((LANE, D_EDGE), jnp.float32),
            pltpu.VMEM_SHARED((N, D_EDGE), jnp.float32),
            pltpu.VMEM_SHARED((N, D_EDGE), jnp.float32),
        ],
        compiler_params=pltpu.CompilerParams(use_tc_tiling_on_sc=False),
    )
    return f(dst1, edge_attr)


BN = 1000


def _tc_body(pagg_ref, pcnt_ref, x_ref, g_ref, we_ref, wx_ref, wg_ref,
             b_ref, o_ref):
    s = pagg_ref[0] + pagg_ref[1]
    cnt = (pcnt_ref[0, :, :1] + pcnt_ref[1, :, :1])
    mean = s / jnp.maximum(cnt, 1.0)
    gw = jnp.dot(g_ref[...], wg_ref[...], preferred_element_type=jnp.float32)
    out = (jnp.dot(mean, we_ref[...], preferred_element_type=jnp.float32)
           + jnp.dot(x_ref[...], wx_ref[...], preferred_element_type=jnp.float32)
           + gw + b_ref[...])
    o_ref[...] = out


@jax.jit
def _tc_combine(pagg, pcnt, x, g2, w_e, w_x, w_g, b2):
    grid = (N // BN,)
    return pl.pallas_call(
        _tc_body,
        grid=grid,
        in_specs=[
            pl.BlockSpec((NUM_CORES, BN, D_EDGE), lambda i: (0, i, 0)),
            pl.BlockSpec((NUM_CORES, BN, D_EDGE), lambda i: (0, i, 0)),
            pl.BlockSpec((BN, D_FEAT), lambda i: (i, 0)),
            pl.BlockSpec((1, D_GLOB), lambda i: (0, 0)),
            pl.BlockSpec((D_EDGE, D_OUT), lambda i: (0, 0)),
            pl.BlockSpec((D_FEAT, D_OUT), lambda i: (0, 0)),
            pl.BlockSpec((D_GLOB, D_OUT), lambda i: (0, 0)),
            pl.BlockSpec((1, D_OUT), lambda i: (0, 0)),
        ],
        out_specs=pl.BlockSpec((BN, D_OUT), lambda i: (i, 0)),
        out_shape=jax.ShapeDtypeStruct((N, D_OUT), jnp.float32),
    )(pagg, pcnt, x, g2, w_e, w_x, w_g, b2)


def kernel(x, edge_index, edge_attr, global_attr, W, b):
    pagg, pcnt = _sc_scatter(edge_index[1], edge_attr)
    w_e = W[:D_EDGE]
    w_x = W[D_EDGE:D_EDGE + D_FEAT]
    w_g = W[D_EDGE + D_FEAT:]
    g2 = global_attr.reshape(1, D_GLOB)
    b2 = b.reshape(1, D_OUT)
    return _tc_combine(pagg, pcnt, x, g2, w_e, w_x, w_g, b2)


# speedup vs baseline: 6.9096x; 1.1480x over previous
import functools

import jax
import jax.numpy as jnp
from jax import lax
from jax.experimental import pallas as pl
from jax.experimental.pallas import tpu as pltpu
from jax.experimental.pallas import tpu_sc as plsc

N = 10000
E = 320000
D_EDGE = 16
D_FEAT = 128
D_GLOB = 128
D_OUT = 128

NUM_CORES = 2
NUM_SUBCORES = 16
NUM_WORKERS = NUM_CORES * NUM_SUBCORES

LANE = 128
ROWS = E // LANE
ROWS_PER_W = ROWS // NUM_WORKERS
ROWS_TAIL = ROWS - ROWS_PER_W * NUM_WORKERS
GROUPS = 6
ROWS_PER_G = ROWS_PER_W // GROUPS
EDGES_PER_G = ROWS_PER_G * LANE
EXP = 624
EXP_TAIL = N - EXP * NUM_SUBCORES


def _sc_scatter_body(dst_hbm, ea_hbm, agg_out, cnt_out,
                     idx_v, ea_v, ones_v, agg_sh, cnt_sh, ldsem, scsem):
    c = lax.axis_index("c")
    s = lax.axis_index("s")
    w = c * NUM_SUBCORES + s

    zrow = jnp.zeros((D_EDGE,), jnp.float32)
    orow = jnp.ones((D_EDGE,), jnp.float32)

    def _fill(i, _):
        ea_v[0, i, :] = zrow
        return 0
    lax.fori_loop(0, EXP, _fill, 0)

    def _fill1(i, _):
        ones_v[i, :] = orow
        return 0
    lax.fori_loop(0, LANE, _fill1, 0)

    r0 = s * EXP
    pltpu.sync_copy(ea_v.at[0, pl.ds(0, EXP), :], agg_sh.at[pl.ds(r0, EXP), :])
    pltpu.sync_copy(ea_v.at[0, pl.ds(0, EXP), :], cnt_sh.at[pl.ds(r0, EXP), :])

    @pl.when(s == 0)
    def _zero_tail():
        t0 = EXP * NUM_SUBCORES
        pltpu.sync_copy(ea_v.at[0, pl.ds(0, EXP_TAIL), :],
                        agg_sh.at[pl.ds(t0, EXP_TAIL), :])
        pltpu.sync_copy(ea_v.at[0, pl.ds(0, EXP_TAIL), :],
                        cnt_sh.at[pl.ds(t0, EXP_TAIL), :])

    plsc.subcore_barrier()

    def _load_group(g, buf):
        row0 = w * ROWS_PER_W + g * ROWS_PER_G
        hs = [pltpu.async_copy(
            ea_hbm.at[pl.ds(row0 * LANE, EDGES_PER_G), :],
            ea_v.at[buf], ldsem)]
        for j in range(ROWS_PER_G):
            hs.append(pltpu.async_copy(
                dst_hbm.at[pl.ds((row0 + j) * LANE, LANE)],
                idx_v.at[buf, j], ldsem))
        return hs

    def _fire_scatters(buf):
        hs = []
        for j in range(ROWS_PER_G):
            idx = idx_v.at[buf, j]
            hs.append(pltpu.async_copy(
                ea_v.at[buf, pl.ds(j * LANE, LANE), :],
                agg_sh.at[idx], scsem, add=True))
            hs.append(pltpu.async_copy(ones_v, cnt_sh.at[idx], scsem,
                                       add=True))
        return hs

    loads = _load_group(0, 0)
    prev = []
    for g in range(GROUPS):
        buf = g % 2
        for h in loads:
            h.wait()
        cur = _fire_scatters(buf)
        for h in prev:
            h.wait()
        prev = cur
        if g + 1 < GROUPS:
            loads = _load_group(g + 1, (g + 1) % 2)
    for h in prev:
        h.wait()

    @pl.when(w < ROWS_TAIL)
    def _tail():
        r = NUM_WORKERS * ROWS_PER_W + w
        pltpu.sync_copy(dst_hbm.at[pl.ds(r * LANE, LANE)], idx_v.at[0, 0])
        pltpu.sync_copy(ea_hbm.at[pl.ds(r * LANE, LANE), :],
                        ea_v.at[0, pl.ds(0, LANE), :])
        pltpu.sync_copy(ea_v.at[0, pl.ds(0, LANE), :],
                        agg_sh.at[idx_v.at[0, 0]], add=True)
        pltpu.sync_copy(ones_v, cnt_sh.at[idx_v.at[0, 0]], add=True)

    plsc.subcore_barrier()

    pltpu.sync_copy(agg_sh.at[pl.ds(r0, EXP), :],
                    agg_out.at[c, pl.ds(r0, EXP), :])
    pltpu.sync_copy(cnt_sh.at[pl.ds(r0, EXP), :],
                    cnt_out.at[c, pl.ds(r0, EXP), :])

    @pl.when(s == 0)
    def _export_tail():
        t0 = EXP * NUM_SUBCORES
        pltpu.sync_copy(agg_sh.at[pl.ds(t0, EXP_TAIL), :],
                        agg_out.at[c, pl.ds(t0, EXP_TAIL), :])
        pltpu.sync_copy(cnt_sh.at[pl.ds(t0, EXP_TAIL), :],
                        cnt_out.at[c, pl.ds(t0, EXP_TAIL), :])


@jax.jit
def _sc_scatter(dst1, edge_attr):
    mesh = plsc.VectorSubcoreMesh(core_axis_name="c", subcore_axis_name="s")
    f = pl.kernel(
        _sc_scatter_body,
        mesh=mesh,
        out_type=[
            jax.ShapeDtypeStruct((NUM_CORES, N, D_EDGE), jnp.float32),
            jax.ShapeDtypeStruct((NUM_CORES, N, D_EDGE), jnp.float32),
        ],
        scratch_types=[
            pltpu.VMEM((2, ROWS_PER_G, LANE), jnp.int32),
            pltpu.VMEM((2, EDGES_PER_G, D_EDGE), jnp.float32),
            pltpu.VMEM((LANE, D_EDGE), jnp.float32),
            pltpu.VMEM_SHARED((N, D_EDGE), jnp.float32),
            pltpu.VMEM_SHARED((N, D_EDGE), jnp.float32),
            pltpu.SemaphoreType.DMA,
            pltpu.SemaphoreType.DMA,
        ],
        compiler_params=pltpu.CompilerParams(use_tc_tiling_on_sc=False),
    )
    return f(dst1, edge_attr)


BN = 1000


def _tc_body(pagg_ref, pcnt_ref, x_ref, g_ref, we_ref, wx_ref, wg_ref,
             b_ref, o_ref):
    s = pagg_ref[0] + pagg_ref[1]
    cnt = (pcnt_ref[0, :, :1] + pcnt_ref[1, :, :1])
    mean = s / jnp.maximum(cnt, 1.0)
    gw = jnp.dot(g_ref[...], wg_ref[...], preferred_element_type=jnp.float32)
    out = (jnp.dot(mean, we_ref[...], preferred_element_type=jnp.float32)
           + jnp.dot(x_ref[...], wx_ref[...], preferred_element_type=jnp.float32)
           + gw + b_ref[...])
    o_ref[...] = out


@jax.jit
def _tc_combine(pagg, pcnt, x, g2, w_e, w_x, w_g, b2):
    grid = (N // BN,)
    return pl.pallas_call(
        _tc_body,
        grid=grid,
        in_specs=[
            pl.BlockSpec((NUM_CORES, BN, D_EDGE), lambda i: (0, i, 0)),
            pl.BlockSpec((NUM_CORES, BN, D_EDGE), lambda i: (0, i, 0)),
            pl.BlockSpec((BN, D_FEAT), lambda i: (i, 0)),
            pl.BlockSpec((1, D_GLOB), lambda i: (0, 0)),
            pl.BlockSpec((D_EDGE, D_OUT), lambda i: (0, 0)),
            pl.BlockSpec((D_FEAT, D_OUT), lambda i: (0, 0)),
            pl.BlockSpec((D_GLOB, D_OUT), lambda i: (0, 0)),
            pl.BlockSpec((1, D_OUT), lambda i: (0, 0)),
        ],
        out_specs=pl.BlockSpec((BN, D_OUT), lambda i: (i, 0)),
        out_shape=jax.ShapeDtypeStruct((N, D_OUT), jnp.float32),
    )(pagg, pcnt, x, g2, w_e, w_x, w_g, b2)


def kernel(x, edge_index, edge_attr, global_attr, W, b):
    pagg, pcnt = _sc_scatter(edge_index[1], edge_attr)
    w_e = W[:D_EDGE]
    w_x = W[D_EDGE:D_EDGE + D_FEAT]
    w_g = W[D_EDGE + D_FEAT:]
    g2 = global_attr.reshape(1, D_GLOB)
    b2 = b.reshape(1, D_OUT)
    return _tc_combine(pagg, pcnt, x, g2, w_e, w_x, w_g, b2)


# speedup vs baseline: 8.1626x; 1.1813x over previous
import functools

import jax
import jax.numpy as jnp
from jax import lax
from jax.experimental import pallas as pl
from jax.experimental.pallas import tpu as pltpu
from jax.experimental.pallas import tpu_sc as plsc

N = 10000
E = 320000
D_EDGE = 16
D_FEAT = 128
D_GLOB = 128
D_OUT = 128

NUM_CORES = 2
NUM_SUBCORES = 16
NUM_WORKERS = NUM_CORES * NUM_SUBCORES

LANE = 128
ROWS = E // LANE
ROWS_PER_W = ROWS // NUM_WORKERS
ROWS_TAIL = ROWS - ROWS_PER_W * NUM_WORKERS
GROUPS = 13
TILES_PER_G = ROWS_PER_W // GROUPS
EXP = 624
EXP_TAIL = N - EXP * NUM_SUBCORES


def _sc_scatter_body(ei_hbm, ea_hbm, agg_out, cnt_out,
                     idx_v, buf_v, rows_v, ones_v, zed_v,
                     agg_sh, cnt_sh, ldsem, scsem):
    c = lax.axis_index("c")
    s = lax.axis_index("s")
    w = c * NUM_SUBCORES + s

    zrow = jnp.zeros((D_EDGE,), jnp.float32)
    orow = jnp.ones((D_EDGE,), jnp.float32)

    def _fill(i, _):
        zed_v[i, :] = zrow
        ones_v[i, :] = orow
        return 0
    lax.fori_loop(0, LANE, _fill, 0)

    r0 = s * EXP
    for kk in range(4):
        pltpu.sync_copy(zed_v, agg_sh.at[pl.ds(r0 + kk * LANE, LANE), :])
        pltpu.sync_copy(zed_v, cnt_sh.at[pl.ds(r0 + kk * LANE, LANE), :])
    pltpu.sync_copy(zed_v.at[pl.ds(0, EXP - 4 * LANE), :],
                    agg_sh.at[pl.ds(r0 + 4 * LANE, EXP - 4 * LANE), :])
    pltpu.sync_copy(zed_v.at[pl.ds(0, EXP - 4 * LANE), :],
                    cnt_sh.at[pl.ds(r0 + 4 * LANE, EXP - 4 * LANE), :])

    @pl.when(s == 0)
    def _zero_tail():
        t0 = EXP * NUM_SUBCORES
        pltpu.sync_copy(zed_v.at[pl.ds(0, EXP_TAIL), :],
                        agg_sh.at[pl.ds(t0, EXP_TAIL), :])
        pltpu.sync_copy(zed_v.at[pl.ds(0, EXP_TAIL), :],
                        cnt_sh.at[pl.ds(t0, EXP_TAIL), :])

    plsc.subcore_barrier()

    iota16 = lax.iota(jnp.int32, D_EDGE)

    def _load_tile(t, k):
        return [
            pltpu.async_copy(ei_hbm.at[t, 1, :], idx_v.at[k % 3], ldsem),
            pltpu.async_copy(ea_hbm.at[0, t], buf_v.at[k % 2, pl.ds(0, 8), :],
                             ldsem),
            pltpu.async_copy(ea_hbm.at[1, t], buf_v.at[k % 2, pl.ds(8, 8), :],
                             ldsem),
        ]

    def _transpose_tile(k):
        bufp = buf_v.at[k % 2]
        slot = k % 2

        def _onerow(e8, _):
            for u in range(8):
                e = e8 * 8 + u
                v = plsc.load_gather(bufp, [iota16, jnp.full((D_EDGE,), e,
                                                             jnp.int32)])
                rows_v[slot, e, :] = v
            return 0
        lax.fori_loop(0, LANE // 8, _onerow, 0)

    def _fire_scatters(k):
        idx = idx_v.at[k % 3]
        return [
            pltpu.async_copy(rows_v.at[k % 2], agg_sh.at[idx], scsem,
                             add=True),
            pltpu.async_copy(ones_v, cnt_sh.at[idx], scsem, add=True),
        ]

    def _group(g, _):
        base = w * ROWS_PER_W + g * TILES_PER_G
        loads = {0: _load_tile(base, 0)}
        scat = {}
        for k in range(TILES_PER_G):
            for h in loads.pop(k):
                h.wait()
            if k >= 2:
                for h in scat.pop(k - 2):
                    h.wait()
            if k + 1 < TILES_PER_G:
                loads[k + 1] = _load_tile(base + k + 1, k + 1)
            _transpose_tile(k)
            scat[k] = _fire_scatters(k)
        for hs in scat.values():
            for h in hs:
                h.wait()
        return 0
    lax.fori_loop(0, GROUPS, _group, 0)

    @pl.when(w < ROWS_TAIL)
    def _tail():
        t = NUM_WORKERS * ROWS_PER_W + w
        for h in _load_tile(t, 0):
            h.wait()
        _transpose_tile(0)
        pltpu.sync_copy(rows_v.at[0], agg_sh.at[idx_v.at[0]], add=True)
        pltpu.sync_copy(ones_v, cnt_sh.at[idx_v.at[0]], add=True)

    plsc.subcore_barrier()

    pltpu.sync_copy(agg_sh.at[pl.ds(r0, EXP), :],
                    agg_out.at[c, pl.ds(r0, EXP), :])
    pltpu.sync_copy(cnt_sh.at[pl.ds(r0, EXP), :],
                    cnt_out.at[c, pl.ds(r0, EXP), :])

    @pl.when(s == 0)
    def _export_tail():
        t0 = EXP * NUM_SUBCORES
        pltpu.sync_copy(agg_sh.at[pl.ds(t0, EXP_TAIL), :],
                        agg_out.at[c, pl.ds(t0, EXP_TAIL), :])
        pltpu.sync_copy(cnt_sh.at[pl.ds(t0, EXP_TAIL), :],
                        cnt_out.at[c, pl.ds(t0, EXP_TAIL), :])


@jax.jit
def _sc_scatter(ei4, eaB):
    mesh = plsc.VectorSubcoreMesh(core_axis_name="c", subcore_axis_name="s")
    f = pl.kernel(
        _sc_scatter_body,
        mesh=mesh,
        out_type=[
            jax.ShapeDtypeStruct((NUM_CORES, N, D_EDGE), jnp.float32),
            jax.ShapeDtypeStruct((NUM_CORES, N, D_EDGE), jnp.float32),
        ],
        scratch_types=[
            pltpu.VMEM((3, LANE), jnp.int32),
            pltpu.VMEM((2, D_EDGE, LANE), jnp.float32),
            pltpu.VMEM((2, LANE, D_EDGE), jnp.float32),
            pltpu.VMEM((LANE, D_EDGE), jnp.float32),
            pltpu.VMEM((LANE, D_EDGE), jnp.float32),
            pltpu.VMEM_SHARED((N, D_EDGE), jnp.float32),
            pltpu.VMEM_SHARED((N, D_EDGE), jnp.float32),
            pltpu.SemaphoreType.DMA,
            pltpu.SemaphoreType.DMA,
        ],
        compiler_params=pltpu.CompilerParams(use_tc_tiling_on_sc=False,
                                             needs_layout_passes=False),
    )
    return f(ei4, eaB)


BN = 1000


def _tc_body(pagg_ref, pcnt_ref, x_ref, g_ref, we_ref, wx_ref, wg_ref,
             b_ref, o_ref):
    s = pagg_ref[0] + pagg_ref[1]
    cnt = (pcnt_ref[0, :, :1] + pcnt_ref[1, :, :1])
    mean = s / jnp.maximum(cnt, 1.0)
    gw = jnp.dot(g_ref[...], wg_ref[...], preferred_element_type=jnp.float32)
    out = (jnp.dot(mean, we_ref[...], preferred_element_type=jnp.float32)
           + jnp.dot(x_ref[...], wx_ref[...], preferred_element_type=jnp.float32)
           + gw + b_ref[...])
    o_ref[...] = out


@jax.jit
def _tc_combine(pagg, pcnt, x, g2, w_e, w_x, w_g, b2):
    grid = (N // BN,)
    return pl.pallas_call(
        _tc_body,
        grid=grid,
        in_specs=[
            pl.BlockSpec((NUM_CORES, BN, D_EDGE), lambda i: (0, i, 0)),
            pl.BlockSpec((NUM_CORES, BN, D_EDGE), lambda i: (0, i, 0)),
            pl.BlockSpec((BN, D_FEAT), lambda i: (i, 0)),
            pl.BlockSpec((1, D_GLOB), lambda i: (0, 0)),
            pl.BlockSpec((D_EDGE, D_OUT), lambda i: (0, 0)),
            pl.BlockSpec((D_FEAT, D_OUT), lambda i: (0, 0)),
            pl.BlockSpec((D_GLOB, D_OUT), lambda i: (0, 0)),
            pl.BlockSpec((1, D_OUT), lambda i: (0, 0)),
        ],
        out_specs=pl.BlockSpec((BN, D_OUT), lambda i: (i, 0)),
        out_shape=jax.ShapeDtypeStruct((N, D_OUT), jnp.float32),
    )(pagg, pcnt, x, g2, w_e, w_x, w_g, b2)


def kernel(x, edge_index, edge_attr, global_attr, W, b):
    ei4 = edge_index.reshape(2, ROWS, LANE).transpose(1, 0, 2)
    eaB = edge_attr.T.reshape(2, 8, ROWS, LANE).swapaxes(1, 2)
    pagg, pcnt = _sc_scatter(ei4, eaB)
    w_e = W[:D_EDGE]
    w_x = W[D_EDGE:D_EDGE + D_FEAT]
    w_g = W[D_EDGE + D_FEAT:]
    g2 = global_attr.reshape(1, D_GLOB)
    b2 = b.reshape(1, D_OUT)
    return _tc_combine(pagg, pcnt, x, g2, w_e, w_x, w_g, b2)


# speedup vs baseline: 11.8003x; 1.4457x over previous
import functools

import jax
import jax.numpy as jnp
from jax import lax
from jax.experimental import pallas as pl
from jax.experimental.pallas import tpu as pltpu
from jax.experimental.pallas import tpu_sc as plsc

N = 10000
E = 320000
D_EDGE = 16
D_FEAT = 128
D_GLOB = 128
D_OUT = 128

NUM_CORES = 2
NUM_SUBCORES = 16
NUM_WORKERS = NUM_CORES * NUM_SUBCORES

LANE = 128
ROWS = E // LANE
ROWS_PER_W = ROWS // NUM_WORKERS
ROWS_TAIL = ROWS - ROWS_PER_W * NUM_WORKERS
GROUPS = 13
TILES_PER_G = ROWS_PER_W // GROUPS
EXP = 624
EXP_TAIL = N - EXP * NUM_SUBCORES


def _sc_scatter_body(ei_hbm, ea_hbm, agg_out, cnt_out,
                     idx_v, buf_v, rows_v, ones_v, zed_v,
                     agg_sh, cnt_sh, ldsem, scsem):
    c = lax.axis_index("c")
    s = lax.axis_index("s")
    w = c * NUM_SUBCORES + s

    zrow = jnp.zeros((D_EDGE,), jnp.float32)
    orow = jnp.ones((D_EDGE,), jnp.float32)

    def _fill(i, _):
        zed_v[i, :] = zrow
        ones_v[i, :] = orow
        return 0
    lax.fori_loop(0, LANE, _fill, 0)

    r0 = s * EXP
    for kk in range(4):
        pltpu.sync_copy(zed_v, agg_sh.at[pl.ds(r0 + kk * LANE, LANE), :])
        pltpu.sync_copy(zed_v, cnt_sh.at[pl.ds(r0 + kk * LANE, LANE), :])
    pltpu.sync_copy(zed_v.at[pl.ds(0, EXP - 4 * LANE), :],
                    agg_sh.at[pl.ds(r0 + 4 * LANE, EXP - 4 * LANE), :])
    pltpu.sync_copy(zed_v.at[pl.ds(0, EXP - 4 * LANE), :],
                    cnt_sh.at[pl.ds(r0 + 4 * LANE, EXP - 4 * LANE), :])

    @pl.when(s == 0)
    def _zero_tail():
        t0 = EXP * NUM_SUBCORES
        pltpu.sync_copy(zed_v.at[pl.ds(0, EXP_TAIL), :],
                        agg_sh.at[pl.ds(t0, EXP_TAIL), :])
        pltpu.sync_copy(zed_v.at[pl.ds(0, EXP_TAIL), :],
                        cnt_sh.at[pl.ds(t0, EXP_TAIL), :])

    plsc.subcore_barrier()

    iota16 = lax.iota(jnp.int32, D_EDGE)

    def _load_tile(t, k):
        return [
            pltpu.async_copy(ei_hbm.at[t, 1, :], idx_v.at[k % 3], ldsem),
            pltpu.async_copy(ea_hbm.at[0, t], buf_v.at[k % 2, pl.ds(0, 8), :],
                             ldsem),
            pltpu.async_copy(ea_hbm.at[1, t], buf_v.at[k % 2, pl.ds(8, 8), :],
                             ldsem),
        ]

    fconst = [jnp.full((D_EDGE,), f, jnp.int32) for f in range(D_EDGE)]

    def _transpose_tile(k):
        bufp = buf_v.at[k % 2]
        rowsp = rows_v.at[k % 2]

        def _oneblk(e8, _):
            ev = e8 * D_EDGE + iota16
            for f in range(D_EDGE):
                v = bufp[f, pl.ds(e8 * D_EDGE, D_EDGE)]
                plsc.store_scatter(rowsp, [ev, fconst[f]], v)
            return 0
        lax.fori_loop(0, LANE // D_EDGE, _oneblk, 0)

    def _fire_scatters(k):
        idx = idx_v.at[k % 3]
        return [
            pltpu.async_copy(rows_v.at[k % 2], agg_sh.at[idx], scsem,
                             add=True),
            pltpu.async_copy(ones_v, cnt_sh.at[idx], scsem, add=True),
        ]

    def _group(g, _):
        base = w * ROWS_PER_W + g * TILES_PER_G
        loads = {0: _load_tile(base, 0)}
        scat = {}
        for k in range(TILES_PER_G):
            for h in loads.pop(k):
                h.wait()
            if k >= 2:
                for h in scat.pop(k - 2):
                    h.wait()
            if k + 1 < TILES_PER_G:
                loads[k + 1] = _load_tile(base + k + 1, k + 1)
            _transpose_tile(k)
            scat[k] = _fire_scatters(k)
        for hs in scat.values():
            for h in hs:
                h.wait()
        return 0
    lax.fori_loop(0, GROUPS, _group, 0)

    @pl.when(w < ROWS_TAIL)
    def _tail():
        t = NUM_WORKERS * ROWS_PER_W + w
        for h in _load_tile(t, 0):
            h.wait()
        _transpose_tile(0)
        pltpu.sync_copy(rows_v.at[0], agg_sh.at[idx_v.at[0]], add=True)
        pltpu.sync_copy(ones_v, cnt_sh.at[idx_v.at[0]], add=True)

    plsc.subcore_barrier()

    pltpu.sync_copy(agg_sh.at[pl.ds(r0, EXP), :],
                    agg_out.at[c, pl.ds(r0, EXP), :])
    pltpu.sync_copy(cnt_sh.at[pl.ds(r0, EXP), :],
                    cnt_out.at[c, pl.ds(r0, EXP), :])

    @pl.when(s == 0)
    def _export_tail():
        t0 = EXP * NUM_SUBCORES
        pltpu.sync_copy(agg_sh.at[pl.ds(t0, EXP_TAIL), :],
                        agg_out.at[c, pl.ds(t0, EXP_TAIL), :])
        pltpu.sync_copy(cnt_sh.at[pl.ds(t0, EXP_TAIL), :],
                        cnt_out.at[c, pl.ds(t0, EXP_TAIL), :])


@jax.jit
def _sc_scatter(ei4, eaB):
    mesh = plsc.VectorSubcoreMesh(core_axis_name="c", subcore_axis_name="s")
    f = pl.kernel(
        _sc_scatter_body,
        mesh=mesh,
        out_type=[
            jax.ShapeDtypeStruct((NUM_CORES, N, D_EDGE), jnp.float32),
            jax.ShapeDtypeStruct((NUM_CORES, N, D_EDGE), jnp.float32),
        ],
        scratch_types=[
            pltpu.VMEM((3, LANE), jnp.int32),
            pltpu.VMEM((2, D_EDGE, LANE), jnp.float32),
            pltpu.VMEM((2, LANE, D_EDGE), jnp.float32),
            pltpu.VMEM((LANE, D_EDGE), jnp.float32),
            pltpu.VMEM((LANE, D_EDGE), jnp.float32),
            pltpu.VMEM_SHARED((N, D_EDGE), jnp.float32),
            pltpu.VMEM_SHARED((N, D_EDGE), jnp.float32),
            pltpu.SemaphoreType.DMA,
            pltpu.SemaphoreType.DMA,
        ],
        compiler_params=pltpu.CompilerParams(use_tc_tiling_on_sc=False,
                                             needs_layout_passes=False),
    )
    return f(ei4, eaB)


BN = 1000


def _tc_body(pagg_ref, pcnt_ref, x_ref, g_ref, we_ref, wx_ref, wg_ref,
             b_ref, o_ref):
    s = pagg_ref[0] + pagg_ref[1]
    cnt = (pcnt_ref[0, :, :1] + pcnt_ref[1, :, :1])
    mean = s / jnp.maximum(cnt, 1.0)
    gw = jnp.dot(g_ref[...], wg_ref[...], preferred_element_type=jnp.float32)
    out = (jnp.dot(mean, we_ref[...], preferred_element_type=jnp.float32)
           + jnp.dot(x_ref[...], wx_ref[...], preferred_element_type=jnp.float32)
           + gw + b_ref[...])
    o_ref[...] = out


@jax.jit
def _tc_combine(pagg, pcnt, x, g2, w_e, w_x, w_g, b2):
    grid = (N // BN,)
    return pl.pallas_call(
        _tc_body,
        grid=grid,
        in_specs=[
            pl.BlockSpec((NUM_CORES, BN, D_EDGE), lambda i: (0, i, 0)),
            pl.BlockSpec((NUM_CORES, BN, D_EDGE), lambda i: (0, i, 0)),
            pl.BlockSpec((BN, D_FEAT), lambda i: (i, 0)),
            pl.BlockSpec((1, D_GLOB), lambda i: (0, 0)),
            pl.BlockSpec((D_EDGE, D_OUT), lambda i: (0, 0)),
            pl.BlockSpec((D_FEAT, D_OUT), lambda i: (0, 0)),
            pl.BlockSpec((D_GLOB, D_OUT), lambda i: (0, 0)),
            pl.BlockSpec((1, D_OUT), lambda i: (0, 0)),
        ],
        out_specs=pl.BlockSpec((BN, D_OUT), lambda i: (i, 0)),
        out_shape=jax.ShapeDtypeStruct((N, D_OUT), jnp.float32),
    )(pagg, pcnt, x, g2, w_e, w_x, w_g, b2)


def kernel(x, edge_index, edge_attr, global_attr, W, b):
    ei4 = edge_index.reshape(2, ROWS, LANE).transpose(1, 0, 2)
    eaB = edge_attr.T.reshape(2, 8, ROWS, LANE).swapaxes(1, 2)
    pagg, pcnt = _sc_scatter(ei4, eaB)
    w_e = W[:D_EDGE]
    w_x = W[D_EDGE:D_EDGE + D_FEAT]
    w_g = W[D_EDGE + D_FEAT:]
    g2 = global_attr.reshape(1, D_GLOB)
    b2 = b.reshape(1, D_OUT)
    return _tc_combine(pagg, pcnt, x, g2, w_e, w_x, w_g, b2)


# speedup vs baseline: 12.5865x; 1.0666x over previous
import functools

import jax
import jax.numpy as jnp
from jax import lax
from jax.experimental import pallas as pl
from jax.experimental.pallas import tpu as pltpu
from jax.experimental.pallas import tpu_sc as plsc

N = 10000
E = 320000
D_EDGE = 16
D_FEAT = 128
D_GLOB = 128
D_OUT = 128

NUM_CORES = 2
NUM_SUBCORES = 16
NUM_WORKERS = NUM_CORES * NUM_SUBCORES

LANE = 128
ROWS = E // LANE
ROWS_PER_W = ROWS // NUM_WORKERS
ROWS_TAIL = ROWS - ROWS_PER_W * NUM_WORKERS
GROUPS = 13
TILES_PER_G = ROWS_PER_W // GROUPS
EXP = 624
EXP_TAIL = N - EXP * NUM_SUBCORES


def _sc_scatter_body(ei_hbm, ea_hbm, agg_out, cnt_out,
                     idx_v, buf_v, rows_v, ones_v, zed_v,
                     agg_sh, cnt_sh, ldsem, scsem):
    c = lax.axis_index("c")
    s = lax.axis_index("s")
    w = c * NUM_SUBCORES + s

    zrow = jnp.zeros((D_EDGE,), jnp.float32)
    orow = jnp.ones((D_EDGE,), jnp.float32)

    def _fill(i, _):
        zed_v[i, :] = zrow
        ones_v[i, :] = orow
        return 0
    lax.fori_loop(0, LANE, _fill, 0)

    r0 = s * EXP
    for kk in range(4):
        pltpu.sync_copy(zed_v, agg_sh.at[pl.ds(r0 + kk * LANE, LANE), :])
        pltpu.sync_copy(zed_v, cnt_sh.at[pl.ds(r0 + kk * LANE, LANE), :])
    pltpu.sync_copy(zed_v.at[pl.ds(0, EXP - 4 * LANE), :],
                    agg_sh.at[pl.ds(r0 + 4 * LANE, EXP - 4 * LANE), :])
    pltpu.sync_copy(zed_v.at[pl.ds(0, EXP - 4 * LANE), :],
                    cnt_sh.at[pl.ds(r0 + 4 * LANE, EXP - 4 * LANE), :])

    @pl.when(s == 0)
    def _zero_tail():
        t0 = EXP * NUM_SUBCORES
        pltpu.sync_copy(zed_v.at[pl.ds(0, EXP_TAIL), :],
                        agg_sh.at[pl.ds(t0, EXP_TAIL), :])
        pltpu.sync_copy(zed_v.at[pl.ds(0, EXP_TAIL), :],
                        cnt_sh.at[pl.ds(t0, EXP_TAIL), :])

    plsc.subcore_barrier()

    iota16 = lax.iota(jnp.int32, D_EDGE)

    def _load_tile(t, k):
        return [
            pltpu.async_copy(ei_hbm.at[t, 1, :], idx_v.at[k % 3], ldsem),
            pltpu.async_copy(ea_hbm.at[0, t], buf_v.at[k % 2, pl.ds(0, 8), :],
                             ldsem),
            pltpu.async_copy(ea_hbm.at[1, t], buf_v.at[k % 2, pl.ds(8, 8), :],
                             ldsem),
        ]

    fconst = [jnp.full((D_EDGE,), f, jnp.int32) for f in range(D_EDGE)]

    def _transpose_tile(k):
        bufp = buf_v.at[k % 2]
        rowsp = rows_v.at[k % 2]
        for e8 in range(LANE // D_EDGE):
            ev = e8 * D_EDGE + iota16
            for f in range(D_EDGE):
                v = bufp[f, pl.ds(e8 * D_EDGE, D_EDGE)]
                plsc.store_scatter(rowsp, [ev, fconst[f]], v)

    def _fire_scatters(k):
        idx = idx_v.at[k % 3]
        return [
            pltpu.async_copy(rows_v.at[k % 2], agg_sh.at[idx], scsem,
                             add=True),
            pltpu.async_copy(ones_v, cnt_sh.at[idx], scsem, add=True),
        ]

    def _group(g, _):
        base = w * ROWS_PER_W + g * TILES_PER_G
        loads = {0: _load_tile(base, 0)}
        scat = {}
        for k in range(TILES_PER_G):
            for h in loads.pop(k):
                h.wait()
            if k >= 2:
                for h in scat.pop(k - 2):
                    h.wait()
            if k + 1 < TILES_PER_G:
                loads[k + 1] = _load_tile(base + k + 1, k + 1)
            _transpose_tile(k)
            scat[k] = _fire_scatters(k)
        for hs in scat.values():
            for h in hs:
                h.wait()
        return 0
    lax.fori_loop(0, GROUPS, _group, 0)

    @pl.when(w < ROWS_TAIL)
    def _tail():
        t = NUM_WORKERS * ROWS_PER_W + w
        for h in _load_tile(t, 0):
            h.wait()
        _transpose_tile(0)
        pltpu.sync_copy(rows_v.at[0], agg_sh.at[idx_v.at[0]], add=True)
        pltpu.sync_copy(ones_v, cnt_sh.at[idx_v.at[0]], add=True)

    plsc.subcore_barrier()

    pltpu.sync_copy(agg_sh.at[pl.ds(r0, EXP), :],
                    agg_out.at[c, pl.ds(r0, EXP), :])
    pltpu.sync_copy(cnt_sh.at[pl.ds(r0, EXP), :],
                    cnt_out.at[c, pl.ds(r0, EXP), :])

    @pl.when(s == 0)
    def _export_tail():
        t0 = EXP * NUM_SUBCORES
        pltpu.sync_copy(agg_sh.at[pl.ds(t0, EXP_TAIL), :],
                        agg_out.at[c, pl.ds(t0, EXP_TAIL), :])
        pltpu.sync_copy(cnt_sh.at[pl.ds(t0, EXP_TAIL), :],
                        cnt_out.at[c, pl.ds(t0, EXP_TAIL), :])


@jax.jit
def _sc_scatter(ei4, eaB):
    mesh = plsc.VectorSubcoreMesh(core_axis_name="c", subcore_axis_name="s")
    f = pl.kernel(
        _sc_scatter_body,
        mesh=mesh,
        out_type=[
            jax.ShapeDtypeStruct((NUM_CORES, N, D_EDGE), jnp.float32),
            jax.ShapeDtypeStruct((NUM_CORES, N, D_EDGE), jnp.float32),
        ],
        scratch_types=[
            pltpu.VMEM((3, LANE), jnp.int32),
            pltpu.VMEM((2, D_EDGE, LANE), jnp.float32),
            pltpu.VMEM((2, LANE, D_EDGE), jnp.float32),
            pltpu.VMEM((LANE, D_EDGE), jnp.float32),
            pltpu.VMEM((LANE, D_EDGE), jnp.float32),
            pltpu.VMEM_SHARED((N, D_EDGE), jnp.float32),
            pltpu.VMEM_SHARED((N, D_EDGE), jnp.float32),
            pltpu.SemaphoreType.DMA,
            pltpu.SemaphoreType.DMA,
        ],
        compiler_params=pltpu.CompilerParams(use_tc_tiling_on_sc=False,
                                             needs_layout_passes=False),
    )
    return f(ei4, eaB)


BN = 1024
BNL = BN * D_EDGE // 128
BN8 = BN // 8


def _tc_body(pagg_ref, pcnt_ref, x_ref, g_ref, we_ref, wx_ref, wg_ref,
             b_ref, o_ref):
    s = pagg_ref[0] + pagg_ref[1]
    c = pcnt_ref[0] + pcnt_ref[1]
    mean = s / jnp.maximum(c, 1.0)
    gwb = (jnp.dot(g_ref[...], wg_ref[...], preferred_element_type=jnp.float32)
           + b_ref[...])
    we = we_ref[...]
    wx = wx_ref[...]
    for j in range(8):
        out_j = (jnp.dot(mean[:, j * D_EDGE:(j + 1) * D_EDGE], we,
                         preferred_element_type=jnp.float32)
                 + jnp.dot(x_ref[:, j, :], wx,
                           preferred_element_type=jnp.float32)
                 + gwb)
        o_ref[:, j, :] = out_j


@jax.jit
def _tc_combine(pagg, pcnt, x3, g2, w_e, w_x, w_g, b2):
    grid = (pl.cdiv(N, BN),)
    out = pl.pallas_call(
        _tc_body,
        grid=grid,
        in_specs=[
            pl.BlockSpec((NUM_CORES, BNL, 128), lambda i: (0, i, 0)),
            pl.BlockSpec((NUM_CORES, BNL, 128), lambda i: (0, i, 0)),
            pl.BlockSpec((BN8, 8, D_FEAT), lambda i: (i, 0, 0)),
            pl.BlockSpec((1, D_GLOB), lambda i: (0, 0)),
            pl.BlockSpec((D_EDGE, D_OUT), lambda i: (0, 0)),
            pl.BlockSpec((D_FEAT, D_OUT), lambda i: (0, 0)),
            pl.BlockSpec((D_GLOB, D_OUT), lambda i: (0, 0)),
            pl.BlockSpec((1, D_OUT), lambda i: (0, 0)),
        ],
        out_specs=pl.BlockSpec((BN8, 8, D_OUT), lambda i: (i, 0, 0)),
        out_shape=jax.ShapeDtypeStruct((N // 8, 8, D_OUT), jnp.float32),
    )(pagg, pcnt, x3, g2, w_e, w_x, w_g, b2)
    return out.reshape(N, D_OUT)


def kernel(x, edge_index, edge_attr, global_attr, W, b):
    ei4 = edge_index.reshape(2, ROWS, LANE).transpose(1, 0, 2)
    eaB = edge_attr.T.reshape(2, 8, ROWS, LANE).swapaxes(1, 2)
    pagg, pcnt = _sc_scatter(ei4, eaB)
    pagg = pagg.reshape(NUM_CORES, N * D_EDGE // 128, 128)
    pcnt = pcnt.reshape(NUM_CORES, N * D_EDGE // 128, 128)
    w_e = W[:D_EDGE]
    w_x = W[D_EDGE:D_EDGE + D_FEAT]
    w_g = W[D_EDGE + D_FEAT:]
    g2 = global_attr.reshape(1, D_GLOB)
    b2 = b.reshape(1, D_OUT)
    x3 = x.reshape(N // 8, 8, D_FEAT)
    return _tc_combine(pagg, pcnt, x3, g2, w_e, w_x, w_g, b2)


# speedup vs baseline: 12.7869x; 1.0159x over previous
import functools

import jax
import jax.numpy as jnp
from jax import lax
from jax.experimental import pallas as pl
from jax.experimental.pallas import tpu as pltpu
from jax.experimental.pallas import tpu_sc as plsc

N = 10000
E = 320000
D_EDGE = 16
D_FEAT = 128
D_GLOB = 128
D_OUT = 128

NUM_CORES = 2
NUM_SUBCORES = 16
NUM_WORKERS = NUM_CORES * NUM_SUBCORES

LANE = 128
ROWS = E // LANE
ROWS_PER_W = ROWS // NUM_WORKERS
ROWS_TAIL = ROWS - ROWS_PER_W * NUM_WORKERS
GROUPS = 13
TILES_PER_G = ROWS_PER_W // GROUPS
EXP = 624
EXP_TAIL = N - EXP * NUM_SUBCORES


def _sc_scatter_body(ei_hbm, ea_hbm, agg_out, cnt_out,
                     idx_v, buf_v, rows_v, ones_v, zed_v,
                     agg_sh, cnt_sh, ldsem, scsem):
    c = lax.axis_index("c")
    s = lax.axis_index("s")
    w = c * NUM_SUBCORES + s

    zrow = jnp.zeros((D_EDGE,), jnp.float32)
    orow = jnp.ones((D_EDGE,), jnp.float32)

    def _fill(i, _):
        zed_v[i, :] = zrow
        ones_v[i, :] = orow
        return 0
    lax.fori_loop(0, LANE, _fill, 0)

    r0 = s * EXP
    for kk in range(4):
        pltpu.sync_copy(zed_v, agg_sh.at[pl.ds(r0 + kk * LANE, LANE), :])
        pltpu.sync_copy(zed_v, cnt_sh.at[pl.ds(r0 + kk * LANE, LANE), :])
    pltpu.sync_copy(zed_v.at[pl.ds(0, EXP - 4 * LANE), :],
                    agg_sh.at[pl.ds(r0 + 4 * LANE, EXP - 4 * LANE), :])
    pltpu.sync_copy(zed_v.at[pl.ds(0, EXP - 4 * LANE), :],
                    cnt_sh.at[pl.ds(r0 + 4 * LANE, EXP - 4 * LANE), :])

    @pl.when(s == 0)
    def _zero_tail():
        t0 = EXP * NUM_SUBCORES
        pltpu.sync_copy(zed_v.at[pl.ds(0, EXP_TAIL), :],
                        agg_sh.at[pl.ds(t0, EXP_TAIL), :])
        pltpu.sync_copy(zed_v.at[pl.ds(0, EXP_TAIL), :],
                        cnt_sh.at[pl.ds(t0, EXP_TAIL), :])

    plsc.subcore_barrier()

    iota16 = lax.iota(jnp.int32, D_EDGE)

    def _load_tile(t, k):
        return [
            pltpu.async_copy(ei_hbm.at[t, 1, :], idx_v.at[k % 3], ldsem),
            pltpu.async_copy(ea_hbm.at[0, t], buf_v.at[k % 2, pl.ds(0, 8), :],
                             ldsem),
            pltpu.async_copy(ea_hbm.at[1, t], buf_v.at[k % 2, pl.ds(8, 8), :],
                             ldsem),
        ]

    fconst = [jnp.full((D_EDGE,), f, jnp.int32) for f in range(D_EDGE)]

    def _transpose_tile(k):
        bufp = buf_v.at[k % 2]
        rowsp = rows_v.at[k % 2]
        for e8 in range(LANE // D_EDGE):
            ev = e8 * D_EDGE + iota16
            vs = [bufp[f, pl.ds(e8 * D_EDGE, D_EDGE)] for f in range(D_EDGE)]
            for f in range(D_EDGE):
                plsc.store_scatter(rowsp, [ev, fconst[f]], vs[f])

    def _fire_scatters(k):
        idx = idx_v.at[k % 3]
        return [
            pltpu.async_copy(rows_v.at[k % 2], agg_sh.at[idx], scsem,
                             add=True),
            pltpu.async_copy(ones_v, cnt_sh.at[idx], scsem, add=True),
        ]

    def _group(g, _):
        base = w * ROWS_PER_W + g * TILES_PER_G
        loads = {0: _load_tile(base, 0)}
        scat = {}
        for k in range(TILES_PER_G):
            for h in loads.pop(k):
                h.wait()
            if k >= 2:
                for h in scat.pop(k - 2):
                    h.wait()
            if k + 1 < TILES_PER_G:
                loads[k + 1] = _load_tile(base + k + 1, k + 1)
            _transpose_tile(k)
            scat[k] = _fire_scatters(k)
        for hs in scat.values():
            for h in hs:
                h.wait()
        return 0
    lax.fori_loop(0, GROUPS, _group, 0)

    @pl.when(w < ROWS_TAIL)
    def _tail():
        t = NUM_WORKERS * ROWS_PER_W + w
        for h in _load_tile(t, 0):
            h.wait()
        _transpose_tile(0)
        pltpu.sync_copy(rows_v.at[0], agg_sh.at[idx_v.at[0]], add=True)
        pltpu.sync_copy(ones_v, cnt_sh.at[idx_v.at[0]], add=True)

    plsc.subcore_barrier()

    pltpu.sync_copy(agg_sh.at[pl.ds(r0, EXP), :],
                    agg_out.at[c, pl.ds(r0, EXP), :])
    pltpu.sync_copy(cnt_sh.at[pl.ds(r0, EXP), :],
                    cnt_out.at[c, pl.ds(r0, EXP), :])

    @pl.when(s == 0)
    def _export_tail():
        t0 = EXP * NUM_SUBCORES
        pltpu.sync_copy(agg_sh.at[pl.ds(t0, EXP_TAIL), :],
                        agg_out.at[c, pl.ds(t0, EXP_TAIL), :])
        pltpu.sync_copy(cnt_sh.at[pl.ds(t0, EXP_TAIL), :],
                        cnt_out.at[c, pl.ds(t0, EXP_TAIL), :])


@jax.jit
def _sc_scatter(ei4, eaB):
    mesh = plsc.VectorSubcoreMesh(core_axis_name="c", subcore_axis_name="s")
    f = pl.kernel(
        _sc_scatter_body,
        mesh=mesh,
        out_type=[
            jax.ShapeDtypeStruct((NUM_CORES, N, D_EDGE), jnp.float32),
            jax.ShapeDtypeStruct((NUM_CORES, N, D_EDGE), jnp.float32),
        ],
        scratch_types=[
            pltpu.VMEM((3, LANE), jnp.int32),
            pltpu.VMEM((2, D_EDGE, LANE), jnp.float32),
            pltpu.VMEM((2, LANE, D_EDGE), jnp.float32),
            pltpu.VMEM((LANE, D_EDGE), jnp.float32),
            pltpu.VMEM((LANE, D_EDGE), jnp.float32),
            pltpu.VMEM_SHARED((N, D_EDGE), jnp.float32),
            pltpu.VMEM_SHARED((N, D_EDGE), jnp.float32),
            pltpu.SemaphoreType.DMA,
            pltpu.SemaphoreType.DMA,
        ],
        compiler_params=pltpu.CompilerParams(use_tc_tiling_on_sc=False,
                                             needs_layout_passes=False),
    )
    return f(ei4, eaB)


BN = 1024
BNL = BN * D_EDGE // 128
BN8 = BN // 8


def _tc_body(pagg_ref, pcnt_ref, x_ref, g_ref, we_ref, wx_ref, wg_ref,
             b_ref, o_ref):
    s = pagg_ref[0] + pagg_ref[1]
    c = pcnt_ref[0] + pcnt_ref[1]
    mean = s / jnp.maximum(c, 1.0)
    gwb = (jnp.dot(g_ref[...], wg_ref[...], preferred_element_type=jnp.float32)
           + b_ref[...])
    we = we_ref[...]
    wx = wx_ref[...]
    for j in range(8):
        out_j = (jnp.dot(mean[:, j * D_EDGE:(j + 1) * D_EDGE], we,
                         preferred_element_type=jnp.float32)
                 + jnp.dot(x_ref[:, j, :], wx,
                           preferred_element_type=jnp.float32)
                 + gwb)
        o_ref[:, j, :] = out_j


@jax.jit
def _tc_combine(pagg, pcnt, x3, g2, w_e, w_x, w_g, b2):
    grid = (pl.cdiv(N, BN),)
    out = pl.pallas_call(
        _tc_body,
        grid=grid,
        in_specs=[
            pl.BlockSpec((NUM_CORES, BNL, 128), lambda i: (0, i, 0)),
            pl.BlockSpec((NUM_CORES, BNL, 128), lambda i: (0, i, 0)),
            pl.BlockSpec((BN8, 8, D_FEAT), lambda i: (i, 0, 0)),
            pl.BlockSpec((1, D_GLOB), lambda i: (0, 0)),
            pl.BlockSpec((D_EDGE, D_OUT), lambda i: (0, 0)),
            pl.BlockSpec((D_FEAT, D_OUT), lambda i: (0, 0)),
            pl.BlockSpec((D_GLOB, D_OUT), lambda i: (0, 0)),
            pl.BlockSpec((1, D_OUT), lambda i: (0, 0)),
        ],
        out_specs=pl.BlockSpec((BN8, 8, D_OUT), lambda i: (i, 0, 0)),
        out_shape=jax.ShapeDtypeStruct((N // 8, 8, D_OUT), jnp.float32),
    )(pagg, pcnt, x3, g2, w_e, w_x, w_g, b2)
    return out.reshape(N, D_OUT)


def kernel(x, edge_index, edge_attr, global_attr, W, b):
    ei4 = edge_index.reshape(2, ROWS, LANE).transpose(1, 0, 2)
    eaB = edge_attr.T.reshape(2, 8, ROWS, LANE).swapaxes(1, 2)
    pagg, pcnt = _sc_scatter(ei4, eaB)
    pagg = pagg.reshape(NUM_CORES, N * D_EDGE // 128, 128)
    pcnt = pcnt.reshape(NUM_CORES, N * D_EDGE // 128, 128)
    w_e = W[:D_EDGE]
    w_x = W[D_EDGE:D_EDGE + D_FEAT]
    w_g = W[D_EDGE + D_FEAT:]
    g2 = global_attr.reshape(1, D_GLOB)
    b2 = b.reshape(1, D_OUT)
    x3 = x.reshape(N // 8, 8, D_FEAT)
    return _tc_combine(pagg, pcnt, x3, g2, w_e, w_x, w_g, b2)


# speedup vs baseline: 13.3240x; 1.0420x over previous
import functools

import jax
import jax.numpy as jnp
from jax import lax
from jax.experimental import pallas as pl
from jax.experimental.pallas import tpu as pltpu
from jax.experimental.pallas import tpu_sc as plsc

N = 10000
E = 320000
D_EDGE = 16
D_FEAT = 128
D_GLOB = 128
D_OUT = 128

NUM_CORES = 2
NUM_SUBCORES = 16
NUM_WORKERS = NUM_CORES * NUM_SUBCORES

LANE = 128
ROWS = E // LANE
ROWS_PER_W = ROWS // NUM_WORKERS
ROWS_TAIL = ROWS - ROWS_PER_W * NUM_WORKERS
GROUPS = 13
TILES_PER_G = ROWS_PER_W // GROUPS
EXP = 624
EXP_TAIL = N - EXP * NUM_SUBCORES


def _sc_scatter_body(ei_hbm, ea_hbm, agg_out, cnt_out,
                     idx_v, buf_v, rows_v, ones_v, zed_v, iota_v,
                     agg_sh, cnt_sh, ldsem, scsem):
    c = lax.axis_index("c")
    s = lax.axis_index("s")
    w = c * NUM_SUBCORES + s

    zrow = jnp.zeros((D_EDGE,), jnp.float32)
    orow = jnp.ones((D_EDGE,), jnp.float32)

    def _fill(i, _):
        zed_v[i, :] = zrow
        ones_v[i, :] = orow
        return 0
    lax.fori_loop(0, LANE, _fill, 0)
    iota_v[...] = lax.iota(jnp.int32, D_EDGE)

    r0 = s * EXP
    for kk in range(4):
        pltpu.sync_copy(zed_v, agg_sh.at[pl.ds(r0 + kk * LANE, LANE), :])
        pltpu.sync_copy(zed_v, cnt_sh.at[pl.ds(r0 + kk * LANE, LANE), :])
    pltpu.sync_copy(zed_v.at[pl.ds(0, EXP - 4 * LANE), :],
                    agg_sh.at[pl.ds(r0 + 4 * LANE, EXP - 4 * LANE), :])
    pltpu.sync_copy(zed_v.at[pl.ds(0, EXP - 4 * LANE), :],
                    cnt_sh.at[pl.ds(r0 + 4 * LANE, EXP - 4 * LANE), :])

    @pl.when(s == 0)
    def _zero_tail():
        t0 = EXP * NUM_SUBCORES
        pltpu.sync_copy(zed_v.at[pl.ds(0, EXP_TAIL), :],
                        agg_sh.at[pl.ds(t0, EXP_TAIL), :])
        pltpu.sync_copy(zed_v.at[pl.ds(0, EXP_TAIL), :],
                        cnt_sh.at[pl.ds(t0, EXP_TAIL), :])

    plsc.subcore_barrier()

    iota16 = lax.iota(jnp.int32, D_EDGE)

    def _load_tile(t, k):
        return [
            pltpu.async_copy(ei_hbm.at[t, 1, :], idx_v.at[k % 3], ldsem),
            pltpu.async_copy(ea_hbm.at[0, t], buf_v.at[k % 2, pl.ds(0, 8), :],
                             ldsem),
            pltpu.async_copy(ea_hbm.at[1, t], buf_v.at[k % 2, pl.ds(8, 8), :],
                             ldsem),
        ]

    def _transpose_tile(k):
        bufp = buf_v.at[k % 2]
        rowsp = rows_v.at[k % 2]
        ebase = iota_v[...]
        for e8 in range(LANE // D_EDGE):
            ev = ebase + e8 * D_EDGE
            vs = [bufp[f, pl.ds(e8 * D_EDGE, D_EDGE)] for f in range(D_EDGE)]
            for f in range(D_EDGE):
                plsc.store_scatter(rowsp, [ev, jnp.full((D_EDGE,), f,
                                                        jnp.int32)], vs[f])

    def _fire_scatters(k):
        idx = idx_v.at[k % 3]
        return [
            pltpu.async_copy(rows_v.at[k % 2], agg_sh.at[idx], scsem,
                             add=True),
            pltpu.async_copy(ones_v, cnt_sh.at[idx], scsem, add=True),
        ]

    def _group(g, _):
        base = w * ROWS_PER_W + g * TILES_PER_G
        loads = {0: _load_tile(base, 0)}
        scat = {}
        for k in range(TILES_PER_G):
            for h in loads.pop(k):
                h.wait()
            if k >= 2:
                for h in scat.pop(k - 2):
                    h.wait()
            if k + 1 < TILES_PER_G:
                loads[k + 1] = _load_tile(base + k + 1, k + 1)
            _transpose_tile(k)
            scat[k] = _fire_scatters(k)
        for hs in scat.values():
            for h in hs:
                h.wait()
        return 0
    lax.fori_loop(0, GROUPS, _group, 0)

    @pl.when(w < ROWS_TAIL)
    def _tail():
        t = NUM_WORKERS * ROWS_PER_W + w
        for h in _load_tile(t, 0):
            h.wait()
        _transpose_tile(0)
        pltpu.sync_copy(rows_v.at[0], agg_sh.at[idx_v.at[0]], add=True)
        pltpu.sync_copy(ones_v, cnt_sh.at[idx_v.at[0]], add=True)

    plsc.subcore_barrier()

    pltpu.sync_copy(agg_sh.at[pl.ds(r0, EXP), :],
                    agg_out.at[c, pl.ds(r0, EXP), :])
    pltpu.sync_copy(cnt_sh.at[pl.ds(r0, EXP), :],
                    cnt_out.at[c, pl.ds(r0, EXP), :])

    @pl.when(s == 0)
    def _export_tail():
        t0 = EXP * NUM_SUBCORES
        pltpu.sync_copy(agg_sh.at[pl.ds(t0, EXP_TAIL), :],
                        agg_out.at[c, pl.ds(t0, EXP_TAIL), :])
        pltpu.sync_copy(cnt_sh.at[pl.ds(t0, EXP_TAIL), :],
                        cnt_out.at[c, pl.ds(t0, EXP_TAIL), :])


@jax.jit
def _sc_scatter(ei4, eaB):
    mesh = plsc.VectorSubcoreMesh(core_axis_name="c", subcore_axis_name="s")
    f = pl.kernel(
        _sc_scatter_body,
        mesh=mesh,
        out_type=[
            jax.ShapeDtypeStruct((NUM_CORES, N, D_EDGE), jnp.float32),
            jax.ShapeDtypeStruct((NUM_CORES, N, D_EDGE), jnp.float32),
        ],
        scratch_types=[
            pltpu.VMEM((3, LANE), jnp.int32),
            pltpu.VMEM((2, D_EDGE, LANE), jnp.float32),
            pltpu.VMEM((2, LANE, D_EDGE), jnp.float32),
            pltpu.VMEM((LANE, D_EDGE), jnp.float32),
            pltpu.VMEM((LANE, D_EDGE), jnp.float32),
            pltpu.VMEM((D_EDGE,), jnp.int32),
            pltpu.VMEM_SHARED((N, D_EDGE), jnp.float32),
            pltpu.VMEM_SHARED((N, D_EDGE), jnp.float32),
            pltpu.SemaphoreType.DMA,
            pltpu.SemaphoreType.DMA,
        ],
        compiler_params=pltpu.CompilerParams(use_tc_tiling_on_sc=False,
                                             needs_layout_passes=False),
    )
    return f(ei4, eaB)


BN = 1024
BNL = BN * D_EDGE // 128
BN8 = BN // 8


def _tc_body(pagg_ref, pcnt_ref, x_ref, g_ref, we_ref, wx_ref, wg_ref,
             b_ref, o_ref):
    s = pagg_ref[0] + pagg_ref[1]
    c = pcnt_ref[0] + pcnt_ref[1]
    mean = s / jnp.maximum(c, 1.0)
    gwb = (jnp.dot(g_ref[...], wg_ref[...], preferred_element_type=jnp.float32)
           + b_ref[...])
    we = we_ref[...]
    wx = wx_ref[...]
    for j in range(8):
        out_j = (jnp.dot(mean[:, j * D_EDGE:(j + 1) * D_EDGE], we,
                         preferred_element_type=jnp.float32)
                 + jnp.dot(x_ref[:, j, :], wx,
                           preferred_element_type=jnp.float32)
                 + gwb)
        o_ref[:, j, :] = out_j


@jax.jit
def _tc_combine(pagg, pcnt, x3, g2, w_e, w_x, w_g, b2):
    grid = (pl.cdiv(N, BN),)
    out = pl.pallas_call(
        _tc_body,
        grid=grid,
        in_specs=[
            pl.BlockSpec((NUM_CORES, BNL, 128), lambda i: (0, i, 0)),
            pl.BlockSpec((NUM_CORES, BNL, 128), lambda i: (0, i, 0)),
            pl.BlockSpec((BN8, 8, D_FEAT), lambda i: (i, 0, 0)),
            pl.BlockSpec((1, D_GLOB), lambda i: (0, 0)),
            pl.BlockSpec((D_EDGE, D_OUT), lambda i: (0, 0)),
            pl.BlockSpec((D_FEAT, D_OUT), lambda i: (0, 0)),
            pl.BlockSpec((D_GLOB, D_OUT), lambda i: (0, 0)),
            pl.BlockSpec((1, D_OUT), lambda i: (0, 0)),
        ],
        out_specs=pl.BlockSpec((BN8, 8, D_OUT), lambda i: (i, 0, 0)),
        out_shape=jax.ShapeDtypeStruct((N // 8, 8, D_OUT), jnp.float32),
    )(pagg, pcnt, x3, g2, w_e, w_x, w_g, b2)
    return out.reshape(N, D_OUT)


def kernel(x, edge_index, edge_attr, global_attr, W, b):
    ei4 = edge_index.reshape(2, ROWS, LANE).transpose(1, 0, 2)
    eaB = edge_attr.T.reshape(2, 8, ROWS, LANE).swapaxes(1, 2)
    pagg, pcnt = _sc_scatter(ei4, eaB)
    pagg = pagg.reshape(NUM_CORES, N * D_EDGE // 128, 128)
    pcnt = pcnt.reshape(NUM_CORES, N * D_EDGE // 128, 128)
    w_e = W[:D_EDGE]
    w_x = W[D_EDGE:D_EDGE + D_FEAT]
    w_g = W[D_EDGE + D_FEAT:]
    g2 = global_attr.reshape(1, D_GLOB)
    b2 = b.reshape(1, D_OUT)
    x3 = x.reshape(N // 8, 8, D_FEAT)
    return _tc_combine(pagg, pcnt, x3, g2, w_e, w_x, w_g, b2)


# speedup vs baseline: 20.8171x; 1.5624x over previous
import functools

import jax
import jax.numpy as jnp
from jax import lax
from jax.experimental import pallas as pl
from jax.experimental.pallas import tpu as pltpu
from jax.experimental.pallas import tpu_sc as plsc

N = 10000
E = 320000
D_EDGE = 16
D_FEAT = 128
D_GLOB = 128
D_OUT = 128

NUM_CORES = 2
NUM_SUBCORES = 16
NUM_WORKERS = NUM_CORES * NUM_SUBCORES

LANE = 128
ROWS = E // LANE
ROWS_PER_W = ROWS // NUM_WORKERS
ROWS_TAIL = ROWS - ROWS_PER_W * NUM_WORKERS
DEP = 6
LAG = 2
SI = 8
EXP = 624
EXP_TAIL = N - EXP * NUM_SUBCORES


def _sc_scatter_body(ei_hbm, ea_hbm, agg_out, cnt_out,
                     idx_v, buf_v, rows_v, ones_v, zed_v, iota_v,
                     agg_sh, cnt_sh, ldsem, scsem):
    c = lax.axis_index("c")
    s = lax.axis_index("s")
    w = c * NUM_SUBCORES + s

    zrow = jnp.zeros((D_EDGE,), jnp.float32)
    orow = jnp.ones((D_EDGE,), jnp.float32)

    def _fill(i, _):
        zed_v[i, :] = zrow
        ones_v[i, :] = orow
        return 0
    lax.fori_loop(0, LANE, _fill, 0)
    iota_v[...] = lax.iota(jnp.int32, D_EDGE)

    r0 = s * EXP
    for kk in range(4):
        pltpu.sync_copy(zed_v, agg_sh.at[pl.ds(r0 + kk * LANE, LANE), :])
        pltpu.sync_copy(zed_v, cnt_sh.at[pl.ds(r0 + kk * LANE, LANE), :])
    pltpu.sync_copy(zed_v.at[pl.ds(0, EXP - 4 * LANE), :],
                    agg_sh.at[pl.ds(r0 + 4 * LANE, EXP - 4 * LANE), :])
    pltpu.sync_copy(zed_v.at[pl.ds(0, EXP - 4 * LANE), :],
                    cnt_sh.at[pl.ds(r0 + 4 * LANE, EXP - 4 * LANE), :])

    @pl.when(s == 0)
    def _zero_tail():
        t0 = EXP * NUM_SUBCORES
        pltpu.sync_copy(zed_v.at[pl.ds(0, EXP_TAIL), :],
                        agg_sh.at[pl.ds(t0, EXP_TAIL), :])
        pltpu.sync_copy(zed_v.at[pl.ds(0, EXP_TAIL), :],
                        cnt_sh.at[pl.ds(t0, EXP_TAIL), :])

    plsc.subcore_barrier()

    iota16 = lax.iota(jnp.int32, D_EDGE)

    def _fire_loads(t, i):
        pltpu.async_copy(ei_hbm.at[t, 1, :], idx_v.at[lax.rem(i, SI)], ldsem)
        pltpu.async_copy(ea_hbm.at[0, t],
                         buf_v.at[lax.rem(i, DEP), pl.ds(0, 8), :], ldsem)
        pltpu.async_copy(ea_hbm.at[1, t],
                         buf_v.at[lax.rem(i, DEP), pl.ds(8, 8), :], ldsem)

    def _drain_loads():
        pltpu.make_async_copy(ei_hbm.at[0, 1, :], idx_v.at[0], ldsem).wait()
        pltpu.make_async_copy(ea_hbm.at[0, 0],
                              buf_v.at[0, pl.ds(0, 8), :], ldsem).wait()
        pltpu.make_async_copy(ea_hbm.at[1, 0],
                              buf_v.at[0, pl.ds(8, 8), :], ldsem).wait()

    def _drain_scats():
        pltpu.make_async_copy(agg_out.at[0, pl.ds(0, LANE), :],
                              rows_v.at[0], scsem).wait()
        pltpu.make_async_copy(agg_out.at[0, pl.ds(0, LANE), :],
                              ones_v, scsem).wait()

    def _transpose_tile(im, rp):
        rowsp = rows_v.at[rp]
        ebase = iota_v[...]
        for e8 in range(LANE // D_EDGE):
            ev = ebase + e8 * D_EDGE
            vs = [buf_v[im, f, pl.ds(e8 * D_EDGE, D_EDGE)]
                  for f in range(D_EDGE)]
            for f in range(D_EDGE):
                plsc.store_scatter(rowsp, [ev, jnp.full((D_EDGE,), f,
                                                        jnp.int32)], vs[f])

    base = w * ROWS_PER_W
    for j in range(DEP - 1):
        _fire_loads(base + j, j)

    def _tile(i, _):
        _drain_loads()

        @pl.when(i >= LAG)
        def _ds():
            _drain_scats()

        @pl.when(i + DEP - 1 < ROWS_PER_W)
        def _fl():
            _fire_loads(base + i + DEP - 1, i + DEP - 1)

        im = lax.rem(i, DEP)
        rp = lax.rem(i, 2)
        _transpose_tile(im, rp)
        idx = idx_v.at[lax.rem(i, SI)]
        pltpu.async_copy(rows_v.at[rp], agg_sh.at[idx], scsem, add=True)
        pltpu.async_copy(ones_v, cnt_sh.at[idx], scsem, add=True)
        return 0
    lax.fori_loop(0, ROWS_PER_W, _tile, 0)
    for _ in range(LAG):
        _drain_scats()

    @pl.when(w < ROWS_TAIL)
    def _tail():
        t = NUM_WORKERS * ROWS_PER_W + w
        pltpu.sync_copy(ei_hbm.at[t, 1, :], idx_v.at[0])
        pltpu.sync_copy(ea_hbm.at[0, t], buf_v.at[0, pl.ds(0, 8), :])
        pltpu.sync_copy(ea_hbm.at[1, t], buf_v.at[0, pl.ds(8, 8), :])
        _transpose_tile(0, 0)
        pltpu.sync_copy(rows_v.at[0], agg_sh.at[idx_v.at[0]], add=True)
        pltpu.sync_copy(ones_v, cnt_sh.at[idx_v.at[0]], add=True)

    plsc.subcore_barrier()

    pltpu.sync_copy(agg_sh.at[pl.ds(r0, EXP), :],
                    agg_out.at[c, pl.ds(r0, EXP), :])
    pltpu.sync_copy(cnt_sh.at[pl.ds(r0, EXP), :],
                    cnt_out.at[c, pl.ds(r0, EXP), :])

    @pl.when(s == 0)
    def _export_tail():
        t0 = EXP * NUM_SUBCORES
        pltpu.sync_copy(agg_sh.at[pl.ds(t0, EXP_TAIL), :],
                        agg_out.at[c, pl.ds(t0, EXP_TAIL), :])
        pltpu.sync_copy(cnt_sh.at[pl.ds(t0, EXP_TAIL), :],
                        cnt_out.at[c, pl.ds(t0, EXP_TAIL), :])


@jax.jit
def _sc_scatter(ei4, eaB):
    mesh = plsc.VectorSubcoreMesh(core_axis_name="c", subcore_axis_name="s")
    f = pl.kernel(
        _sc_scatter_body,
        mesh=mesh,
        out_type=[
            jax.ShapeDtypeStruct((NUM_CORES, N, D_EDGE), jnp.float32),
            jax.ShapeDtypeStruct((NUM_CORES, N, D_EDGE), jnp.float32),
        ],
        scratch_types=[
            pltpu.VMEM((SI, LANE), jnp.int32),
            pltpu.VMEM((DEP, D_EDGE, LANE), jnp.float32),
            pltpu.VMEM((2, LANE, D_EDGE), jnp.float32),
            pltpu.VMEM((LANE, D_EDGE), jnp.float32),
            pltpu.VMEM((LANE, D_EDGE), jnp.float32),
            pltpu.VMEM((D_EDGE,), jnp.int32),
            pltpu.VMEM_SHARED((N, D_EDGE), jnp.float32),
            pltpu.VMEM_SHARED((N, D_EDGE), jnp.float32),
            pltpu.SemaphoreType.DMA,
            pltpu.SemaphoreType.DMA,
        ],
        compiler_params=pltpu.CompilerParams(use_tc_tiling_on_sc=False,
                                             needs_layout_passes=False),
    )
    return f(ei4, eaB)


BN = 1024
BNL = BN * D_EDGE // 128
BN8 = BN // 8


def _tc_body(pagg_ref, pcnt_ref, x_ref, g_ref, we_ref, wx_ref, wg_ref,
             b_ref, o_ref):
    s = pagg_ref[0] + pagg_ref[1]
    c = pcnt_ref[0] + pcnt_ref[1]
    mean = s / jnp.maximum(c, 1.0)
    gwb = (jnp.dot(g_ref[...], wg_ref[...], preferred_element_type=jnp.float32)
           + b_ref[...])
    we = we_ref[...]
    wx = wx_ref[...]
    for j in range(8):
        out_j = (jnp.dot(mean[:, j * D_EDGE:(j + 1) * D_EDGE], we,
                         preferred_element_type=jnp.float32)
                 + jnp.dot(x_ref[:, j, :], wx,
                           preferred_element_type=jnp.float32)
                 + gwb)
        o_ref[:, j, :] = out_j


@jax.jit
def _tc_combine(pagg, pcnt, x3, g2, w_e, w_x, w_g, b2):
    grid = (pl.cdiv(N, BN),)
    out = pl.pallas_call(
        _tc_body,
        grid=grid,
        in_specs=[
            pl.BlockSpec((NUM_CORES, BNL, 128), lambda i: (0, i, 0)),
            pl.BlockSpec((NUM_CORES, BNL, 128), lambda i: (0, i, 0)),
            pl.BlockSpec((BN8, 8, D_FEAT), lambda i: (i, 0, 0)),
            pl.BlockSpec((1, D_GLOB), lambda i: (0, 0)),
            pl.BlockSpec((D_EDGE, D_OUT), lambda i: (0, 0)),
            pl.BlockSpec((D_FEAT, D_OUT), lambda i: (0, 0)),
            pl.BlockSpec((D_GLOB, D_OUT), lambda i: (0, 0)),
            pl.BlockSpec((1, D_OUT), lambda i: (0, 0)),
        ],
        out_specs=pl.BlockSpec((BN8, 8, D_OUT), lambda i: (i, 0, 0)),
        out_shape=jax.ShapeDtypeStruct((N // 8, 8, D_OUT), jnp.float32),
    )(pagg, pcnt, x3, g2, w_e, w_x, w_g, b2)
    return out.reshape(N, D_OUT)


def kernel(x, edge_index, edge_attr, global_attr, W, b):
    ei4 = edge_index.reshape(2, ROWS, LANE).transpose(1, 0, 2)
    eaB = edge_attr.T.reshape(2, 8, ROWS, LANE).swapaxes(1, 2)
    pagg, pcnt = _sc_scatter(ei4, eaB)
    pagg = pagg.reshape(NUM_CORES, N * D_EDGE // 128, 128)
    pcnt = pcnt.reshape(NUM_CORES, N * D_EDGE // 128, 128)
    w_e = W[:D_EDGE]
    w_x = W[D_EDGE:D_EDGE + D_FEAT]
    w_g = W[D_EDGE + D_FEAT:]
    g2 = global_attr.reshape(1, D_GLOB)
    b2 = b.reshape(1, D_OUT)
    x3 = x.reshape(N // 8, 8, D_FEAT)
    return _tc_combine(pagg, pcnt, x3, g2, w_e, w_x, w_g, b2)
